# Initial kernel scaffold; baseline (speedup 1.0000x reference)
import jax
import jax.numpy as jnp
from jax.experimental import pallas as pl


def kernel(seq, node_s, edge_index, W_s, W0, al0, ar0, b0, W1, al1, ar1, b1, W2, al2, ar2, b2, Wd1, bd1, Wd2, bd2):
    raise NotImplementedError("write your pallas kernel here")


# speedup vs baseline: 1.0533x; 1.0533x over previous
import functools

import jax
import jax.numpy as jnp
import numpy as np
from jax import lax
from jax.experimental import pallas as pl
from jax.experimental.pallas import tpu as pltpu

N = 50000
E = 800000
H = 4
D = 32
HD = H * D

BN = 1000
NB = N // BN


def _front_body(seq_ref, ns_ref, wsa_ref, wna_ref, feat_ref, elr_ref):
    s = seq_ref[:, 0]
    oh = (s[:, None] == lax.broadcasted_iota(jnp.int32, (BN, 32), 1)).astype(jnp.float32)
    x = oh @ wsa_ref[...] + ns_ref[...] @ wna_ref[...]
    feat_ref[...] = x[:, :HD]
    elr_ref[...] = x[:, HD:]


def _mid_body(h_ref, wa_ref, feat_ref, elr_ref):
    x = h_ref[...] @ wa_ref[...]
    feat_ref[...] = x[:, :HD]
    elr_ref[...] = x[:, HD:]


def _head_body(h_ref, wd1_ref, bd1_ref, wd2_ref, out_ref, acc_ref):
    h = jnp.maximum(h_ref[...], 0.0)
    y = jnp.maximum(h @ wd1_ref[...] + bd1_ref[...], 0.0)
    o = y @ wd2_ref[...] + 0.5
    out_ref[...] = o
    i = pl.program_id(0)

    @pl.when(i == 0)
    def _():
        acc_ref[...] = jnp.zeros_like(acc_ref)

    acc_ref[...] += jnp.sum(o).reshape(1, 1)


def _tc_front(seq2, node_s8, Wsa, Wna):
    return pl.pallas_call(
        _front_body,
        grid=(NB,),
        in_specs=[
            pl.BlockSpec((BN, 1), lambda i: (i, 0)),
            pl.BlockSpec((BN, 8), lambda i: (i, 0)),
            pl.BlockSpec((32, 144), lambda i: (0, 0)),
            pl.BlockSpec((8, 144), lambda i: (0, 0)),
        ],
        out_specs=[
            pl.BlockSpec((BN, HD), lambda i: (i, 0)),
            pl.BlockSpec((BN, 16), lambda i: (i, 0)),
        ],
        out_shape=[
            jax.ShapeDtypeStruct((N, HD), jnp.float32),
            jax.ShapeDtypeStruct((N, 16), jnp.float32),
        ],
    )(seq2, node_s8, Wsa, Wna)


def _tc_mid(h, Wa):
    return pl.pallas_call(
        _mid_body,
        grid=(NB,),
        in_specs=[
            pl.BlockSpec((BN, HD), lambda i: (i, 0)),
            pl.BlockSpec((HD, 144), lambda i: (0, 0)),
        ],
        out_specs=[
            pl.BlockSpec((BN, HD), lambda i: (i, 0)),
            pl.BlockSpec((BN, 16), lambda i: (i, 0)),
        ],
        out_shape=[
            jax.ShapeDtypeStruct((N, HD), jnp.float32),
            jax.ShapeDtypeStruct((N, 16), jnp.float32),
        ],
    )(h, Wa)


def _tc_head(h, Wd1, bd1, Wd2b):
    out, acc = pl.pallas_call(
        _head_body,
        grid=(NB,),
        in_specs=[
            pl.BlockSpec((BN, HD), lambda i: (i, 0)),
            pl.BlockSpec((HD, 512), lambda i: (0, 0)),
            pl.BlockSpec((1, 512), lambda i: (0, 0)),
            pl.BlockSpec((512, 1), lambda i: (0, 0)),
        ],
        out_specs=[
            pl.BlockSpec((BN, 1), lambda i: (i, 0)),
            pl.BlockSpec((1, 1), lambda i: (0, 0)),
        ],
        out_shape=[
            jax.ShapeDtypeStruct((N, 1), jnp.float32),
            jax.ShapeDtypeStruct((1, 1), jnp.float32),
        ],
    )(h, Wd1, bd1, Wd2b)
    return out, acc / N


def _edge_softmax_agg(feat, elr, src, dst, b):
    el = elr[:, :H]
    er = elr[:, H:2 * H]
    x = el[src] + er[dst]
    e = jnp.maximum(x, 0.2 * x)
    ee = jnp.exp(e)
    esum = jax.ops.segment_sum(ee, dst, num_segments=N)
    feath = feat.reshape(N, H, D)
    out = jax.ops.segment_sum(ee[:, :, None] * feath[src], dst, num_segments=N)
    inv = jnp.where(esum > 0, 1.0 / esum, 0.0)
    out = out * inv[:, :, None] + b.reshape(1, H, D)
    return out.reshape(N, HD)


def _fold_aug(W, al, ar):
    Wh = W.reshape(W.shape[0], H, D)
    Wel = jnp.einsum('khd,hd->kh', Wh, al)
    Wer = jnp.einsum('khd,hd->kh', Wh, ar)
    pad = jnp.zeros((W.shape[0], 8), W.dtype)
    return jnp.concatenate([W, Wel, Wer, pad], axis=1)


def kernel(seq, node_s, edge_index, W_s, W0, al0, ar0, b0, W1, al1, ar1, b1,
           W2, al2, ar2, b2, Wd1, bd1, Wd2, bd2):
    src = edge_index[0]
    dst = edge_index[1]

    W0a = _fold_aug(W0, al0, ar0)
    Wsa = jnp.zeros((32, 144), jnp.float32).at[:20].set(W_s @ W0a[:20])
    Wna = W0a[20:26]
    Wna = jnp.concatenate([Wna, jnp.zeros((2, 144), jnp.float32)], axis=0)
    W1a = _fold_aug(W1, al1, ar1)
    W2a = _fold_aug(W2, al2, ar2)

    seq2 = seq.astype(jnp.int32).reshape(N, 1)
    node_s8 = jnp.concatenate([node_s, jnp.zeros((N, 2), jnp.float32)], axis=1)

    feat, elr = _tc_front(seq2, node_s8, Wsa, Wna)
    h = _edge_softmax_agg(feat, elr, src, dst, b0)
    feat, elr = _tc_mid(h, W1a)
    h = _edge_softmax_agg(feat, elr, src, dst, b1)
    feat, elr = _tc_mid(h, W2a)
    h = _edge_softmax_agg(feat, elr, src, dst, b2)

    Wd2b = Wd2 + jnp.zeros((512, 1), jnp.float32)
    out, gmean = _tc_head(h, Wd1, bd1.reshape(1, 512), Wd2b)
    out = out + bd2.reshape(1, 1)
    gmean = gmean + bd2.reshape(1, 1)
    return out, gmean
